# Initial kernel scaffold; baseline (speedup 1.0000x reference)
import jax
import jax.numpy as jnp
from jax.experimental import pallas as pl


def kernel(x, qkv_w, qkv_b, proj_w, proj_b):
    raise NotImplementedError("write your pallas kernel here")


# speedup vs baseline: 5.1673x; 5.1673x over previous
import functools
import math

import jax
import jax.numpy as jnp
from jax import lax
from jax.experimental import pallas as pl
from jax.experimental.pallas import tpu as pltpu

_VMEM_LIMIT = 48 * 1024 * 1024


def _fused_attn_kernel(x_ref, wqkv_ref, bqkv_ref, wproj_ref, bproj_ref,
                       o_ref, *, num_heads, head_dim, scale):
    C = num_heads * head_dim
    xb = x_ref[0].astype(jnp.bfloat16)

    qkv = lax.dot_general(xb, wqkv_ref[...], (((1,), (0,)), ((), ())),
                          preferred_element_type=jnp.float32)
    qkv = qkv + bqkv_ref[...]

    head_outs = []
    for h in range(num_heads):
        lo = h * head_dim
        qh = (qkv[:, lo:lo + head_dim] * scale).astype(jnp.bfloat16)
        kh = qkv[:, C + lo:C + lo + head_dim].astype(jnp.bfloat16)
        vh = qkv[:, 2 * C + lo:2 * C + lo + head_dim].astype(jnp.bfloat16)

        s = lax.dot_general(qh, kh, (((1,), (1,)), ((), ())),
                            preferred_element_type=jnp.float32)
        m = jnp.max(s, axis=-1, keepdims=True)
        p = jnp.exp(s - m)
        l = jnp.sum(p, axis=-1, keepdims=True)
        ph = p.astype(jnp.bfloat16)
        oh = lax.dot_general(ph, vh, (((1,), (0,)), ((), ())),
                             preferred_element_type=jnp.float32)
        head_outs.append(oh / l)

    attn = jnp.concatenate(head_outs, axis=1).astype(jnp.bfloat16)

    out = lax.dot_general(attn, wproj_ref[...], (((1,), (0,)), ((), ())),
                          preferred_element_type=jnp.float32)
    o_ref[0] = out + bproj_ref[...]


def kernel(x, qkv_w, qkv_b, proj_w, proj_b):
    B, N, C = x.shape
    num_heads = 12
    head_dim = C // num_heads
    scale = 1.0 / math.sqrt(head_dim)

    wqkv = qkv_w.T.astype(jnp.bfloat16)
    wproj = proj_w.T.astype(jnp.bfloat16)
    bqkv = qkv_b.reshape(1, 3 * C)
    bproj = proj_b.reshape(1, C)

    itemsize = x.dtype.itemsize
    cost = pl.CostEstimate(
        flops=2 * B * N * C * 3 * C + 4 * B * num_heads * N * N * head_dim
              + 2 * B * N * C * C,
        transcendentals=B * num_heads * N * N,
        bytes_accessed=(2 * B * N * C) * itemsize + (3 * C * C + C * C) * 2)

    kern = functools.partial(_fused_attn_kernel, num_heads=num_heads,
                             head_dim=head_dim, scale=scale)
    out = pl.pallas_call(
        kern,
        out_shape=jax.ShapeDtypeStruct((B, N, C), x.dtype),
        grid=(B,),
        in_specs=[
            pl.BlockSpec((1, N, C), lambda b: (b, 0, 0)),
            pl.BlockSpec((C, 3 * C), lambda b: (0, 0)),
            pl.BlockSpec((1, 3 * C), lambda b: (0, 0)),
            pl.BlockSpec((C, C), lambda b: (0, 0)),
            pl.BlockSpec((1, C), lambda b: (0, 0)),
        ],
        out_specs=pl.BlockSpec((1, N, C), lambda b: (b, 0, 0)),
        compiler_params=pltpu.CompilerParams(
            dimension_semantics=("parallel",),
            vmem_limit_bytes=_VMEM_LIMIT),
        cost_estimate=cost,
    )(x, wqkv, bqkv, wproj, bproj)
    return out
